# Initial kernel scaffold; baseline (speedup 1.0000x reference)
import jax
import jax.numpy as jnp
from jax.experimental import pallas as pl


def kernel(edge_index, edge_type, comp1, bases1, root1, bias1, comp2, bases2, root2, bias2, comp3, bases3, root3, bias3):
    raise NotImplementedError("write your pallas kernel here")


# speedup vs baseline: 10.3920x; 10.3920x over previous
import functools

import jax
import jax.numpy as jnp
from jax import lax
from jax.experimental import pallas as pl
from jax.experimental.pallas import tpu as pltpu
from jax.experimental.pallas import tpu_sc as plsc

_N = 10000
_E = 320000
_R = 8
_NB = 4
_C = 128

_NC = 2
_NS = 16
_NW = _NC * _NS
_EPW = _E // _NW
_B = 80
_NBATCH = _EPW // _B
_EPT = _E // _NS
_NCB = _EPT // _B
_KPAD = 81920
_KSL = _KPAD // _NS

_BN = 1000


def _mesh():
    return plsc.VectorSubcoreMesh(core_axis_name="c", subcore_axis_name="s")


def _sc_prologue(src, dst, edge_type):
    @functools.partial(
        pl.kernel,
        out_type=[jax.ShapeDtypeStruct((_E,), jnp.int32),
                  jax.ShapeDtypeStruct((_E,), jnp.float32)],
        mesh=_mesh(),
        scratch_types=[
            pltpu.VMEM((_B,), jnp.int32),
            pltpu.VMEM((_B,), jnp.int32),
            pltpu.VMEM((_B,), jnp.int32),
            pltpu.VMEM((_B,), jnp.int32),
            pltpu.VMEM((_B,), jnp.int32),
            pltpu.VMEM((_B,), jnp.float32),
            pltpu.VMEM((_B,), jnp.float32),
            pltpu.VMEM((_KSL,), jnp.float32),
            pltpu.VMEM_SHARED((_KPAD,), jnp.float32),
            pltpu.SemaphoreType.DMA,
        ],
    )
    def kfn(src_h, dst_h, et, g13_o, w_o,
            src_b, dst_b, typ_b, key_b, g13_b, ones_b, w_b, sbuf, cnt_sh, sem):
        c = lax.axis_index("c")
        s = lax.axis_index("s")
        wid = s * _NC + c

        def zfill(i, carry):
            sbuf[pl.ds(i * 16, 16)] = jnp.zeros((16,), jnp.float32)
            return carry
        lax.fori_loop(0, _KSL // 16, zfill, 0)
        for j in range(_B // 16):
            ones_b[pl.ds(j * 16, 16)] = jnp.ones((16,), jnp.float32)
        pltpu.sync_copy(sbuf, cnt_sh.at[pl.ds(s * _KSL, _KSL)])
        plsc.subcore_barrier()

        def cbody(ib, carry):
            base = s * _EPT + ib * _B
            pltpu.sync_copy(dst_h.at[pl.ds(base, _B)], dst_b)
            pltpu.sync_copy(et.at[pl.ds(base, _B)], typ_b)
            for j in range(_B // 16):
                d = dst_b[pl.ds(j * 16, 16)]
                t = typ_b[pl.ds(j * 16, 16)]
                key_b[pl.ds(j * 16, 16)] = d * _R + t
            pltpu.sync_copy(ones_b, cnt_sh.at[key_b], add=True)
            return carry
        lax.fori_loop(0, _NCB, cbody, 0)
        plsc.subcore_barrier()

        pltpu.sync_copy(cnt_sh.at[pl.ds(s * _KSL, _KSL)], sbuf)

        def rbody(i, carry):
            x = sbuf[pl.ds(i * 16, 16)]
            sbuf[pl.ds(i * 16, 16)] = 1.0 / jnp.maximum(x, 1.0)
            return carry
        lax.fori_loop(0, _KSL // 16, rbody, 0)
        pltpu.sync_copy(sbuf, cnt_sh.at[pl.ds(s * _KSL, _KSL)])
        plsc.subcore_barrier()

        def p2body(ib, carry):
            base = wid * _EPW + ib * _B
            pltpu.sync_copy(src_h.at[pl.ds(base, _B)], src_b)
            pltpu.sync_copy(dst_h.at[pl.ds(base, _B)], dst_b)
            pltpu.sync_copy(et.at[pl.ds(base, _B)], typ_b)
            for j in range(_B // 16):
                sj = src_b[pl.ds(j * 16, 16)]
                dj = dst_b[pl.ds(j * 16, 16)]
                tj = typ_b[pl.ds(j * 16, 16)]
                key_b[pl.ds(j * 16, 16)] = dj * _R + tj
                g13_b[pl.ds(j * 16, 16)] = tj * _N + sj
            pltpu.async_copy(cnt_sh.at[key_b], w_b, sem).wait()
            pltpu.sync_copy(w_b, w_o.at[pl.ds(base, _B)])
            pltpu.sync_copy(g13_b, g13_o.at[pl.ds(base, _B)])
            return carry
        lax.fori_loop(0, _NBATCH, p2body, 0)

    return kfn(src, dst, edge_type)


def _sc_layer(table, g13, dst, w):
    @functools.partial(
        pl.kernel,
        out_type=jax.ShapeDtypeStruct((_NC, _N, _C), jnp.float32),
        mesh=_mesh(),
        scratch_types=[
            pltpu.VMEM((_B,), jnp.int32),
            pltpu.VMEM((_B,), jnp.int32),
            pltpu.VMEM((_B + 16,), jnp.float32),
            pltpu.VMEM((_B, _C), jnp.float32),
            pltpu.VMEM((8, _C), jnp.float32),
            pltpu.VMEM_SHARED((_NS * 632, _C), jnp.float32),
            pltpu.SemaphoreType.DMA,
        ],
    )
    def kfn(table_h, g13_h, dst_h, w_h, out_h,
            gidx_b, dst_b, w_b, rows, zbuf, acc, sem):
        c = lax.axis_index("c")
        s = lax.axis_index("s")
        wid = s * _NC + c
        row_base = s * 632
        nchunks = lax.select(s == _NS - 1, 520 // 8, 632 // 8)

        for i in range(8):
            for j in range(_C // 16):
                zbuf[i, pl.ds(j * 16, 16)] = jnp.zeros((16,), jnp.float32)

        def zcopy(k, carry):
            pltpu.sync_copy(zbuf, acc.at[pl.ds(row_base + k * 8, 8), :])
            return carry
        lax.fori_loop(0, nchunks, zcopy, 0)
        plsc.subcore_barrier()

        def body(ib, carry):
            base = wid * _EPW + ib * _B
            pltpu.sync_copy(g13_h.at[pl.ds(base, _B)], gidx_b)
            pltpu.sync_copy(dst_h.at[pl.ds(base, _B)], dst_b)
            pltpu.sync_copy(w_h.at[pl.ds(base, _B)], w_b.at[pl.ds(0, _B)])
            pltpu.async_copy(table_h.at[gidx_b], rows, sem).wait()

            def scale(e, carry2):
                wv = w_b[pl.ds(e, 16)][0]
                for j in range(_C // 16):
                    rows[e, pl.ds(j * 16, 16)] = rows[e, pl.ds(j * 16, 16)] * wv
                return carry2
            lax.fori_loop(0, _B, scale, 0)
            pltpu.sync_copy(rows, acc.at[dst_b], add=True)
            return carry
        lax.fori_loop(0, _NBATCH, body, 0)
        plsc.subcore_barrier()

        @pl.when(s != _NS - 1)
        def _drain_full():
            pltpu.sync_copy(acc.at[pl.ds(row_base, 632), :],
                            out_h.at[c, pl.ds(row_base, 632), :])

        @pl.when(s == _NS - 1)
        def _drain_tail():
            pltpu.sync_copy(acc.at[pl.ds(15 * 632, 520), :],
                            out_h.at[c, pl.ds(15 * 632, 520), :])

    return kfn(table, g13, dst, w)


def _tc_table1(comp1, bases1):
    def body(cm_ref, bb_ref, o_ref):
        cm = cm_ref[...]
        for r in range(_R):
            acc = cm[r, 0] * bb_ref[0]
            for b in range(1, _NB):
                acc = acc + cm[r, b] * bb_ref[b]
            o_ref[r] = acc
    return pl.pallas_call(
        body,
        grid=(_N // _BN,),
        in_specs=[
            pl.BlockSpec((_R, _NB), lambda i: (0, 0)),
            pl.BlockSpec((_NB, _BN, _C), lambda i: (0, i, 0)),
        ],
        out_specs=pl.BlockSpec((_R, _BN, _C), lambda i: (0, i, 0)),
        out_shape=jax.ShapeDtypeStruct((_R, _N, _C), jnp.float32),
    )(comp1, bases1)


def _tc_layer2(sc1, root1, bias1, comp2, bases2):
    def body(sc_ref, rt_ref, bs_ref, cm_ref, bb_ref, h1_ref, xt_ref):
        h1 = jnp.maximum(sc_ref[0] + sc_ref[1] + rt_ref[...] + bs_ref[...], 0.0)
        h1_ref[...] = h1
        cm = cm_ref[...]
        for r in range(_R):
            wr = cm[r, 0] * bb_ref[0]
            for b in range(1, _NB):
                wr = wr + cm[r, b] * bb_ref[b]
            xt_ref[r] = jnp.dot(h1, wr, preferred_element_type=jnp.float32)
    return pl.pallas_call(
        body,
        grid=(_N // _BN,),
        in_specs=[
            pl.BlockSpec((_NC, _BN, _C), lambda i: (0, i, 0)),
            pl.BlockSpec((_BN, _C), lambda i: (i, 0)),
            pl.BlockSpec((1, _C), lambda i: (0, 0)),
            pl.BlockSpec((_R, _NB), lambda i: (0, 0)),
            pl.BlockSpec((_NB, _C, _C), lambda i: (0, 0, 0)),
        ],
        out_specs=[
            pl.BlockSpec((_BN, _C), lambda i: (i, 0)),
            pl.BlockSpec((_R, _BN, _C), lambda i: (0, i, 0)),
        ],
        out_shape=[
            jax.ShapeDtypeStruct((_N, _C), jnp.float32),
            jax.ShapeDtypeStruct((_R, _N, _C), jnp.float32),
        ],
    )(sc1, root1, bias1, comp2, bases2)


def _tc_layer3(sc2, h1, root2, bias2, comp3, bases3):
    def body(sc_ref, h1_ref, rt_ref, bs_ref, cm_ref, bb_ref, h2_ref, xt_ref):
        h1 = h1_ref[...]
        h2 = jnp.maximum(
            sc_ref[0] + sc_ref[1]
            + jnp.dot(h1, rt_ref[...], preferred_element_type=jnp.float32)
            + bs_ref[...], 0.0)
        h2_ref[...] = h2
        f2 = jnp.concatenate([h1, h2], axis=-1)
        cm = cm_ref[...]
        for r in range(_R):
            wr = cm[r, 0] * bb_ref[0]
            for b in range(1, _NB):
                wr = wr + cm[r, b] * bb_ref[b]
            xt_ref[r] = jnp.dot(f2, wr, preferred_element_type=jnp.float32)
    return pl.pallas_call(
        body,
        grid=(_N // _BN,),
        in_specs=[
            pl.BlockSpec((_NC, _BN, _C), lambda i: (0, i, 0)),
            pl.BlockSpec((_BN, _C), lambda i: (i, 0)),
            pl.BlockSpec((_C, _C), lambda i: (0, 0)),
            pl.BlockSpec((1, _C), lambda i: (0, 0)),
            pl.BlockSpec((_R, _NB), lambda i: (0, 0)),
            pl.BlockSpec((_NB, 2 * _C, _C), lambda i: (0, 0, 0)),
        ],
        out_specs=[
            pl.BlockSpec((_BN, _C), lambda i: (i, 0)),
            pl.BlockSpec((_R, _BN, _C), lambda i: (0, i, 0)),
        ],
        out_shape=[
            jax.ShapeDtypeStruct((_N, _C), jnp.float32),
            jax.ShapeDtypeStruct((_R, _N, _C), jnp.float32),
        ],
    )(sc2, h1, root2, bias2, comp3, bases3)


def _tc_final(sc3, h1, h2, root3, bias3):
    def body(sc_ref, h1_ref, h2_ref, rt_ref, bs_ref, o_ref):
        h1 = h1_ref[...]
        h2 = h2_ref[...]
        rt = rt_ref[...]
        h3 = jnp.maximum(
            sc_ref[0] + sc_ref[1]
            + jnp.dot(h1, rt[:_C], preferred_element_type=jnp.float32)
            + jnp.dot(h2, rt[_C:], preferred_element_type=jnp.float32)
            + bs_ref[...], 0.0)
        f3 = jnp.concatenate([h1, h2, h3], axis=-1)
        m = jnp.max(f3, axis=-1, keepdims=True)
        lse = jnp.log(jnp.sum(jnp.exp(f3 - m), axis=-1, keepdims=True)) + m
        o_ref[...] = f3 - lse
    return pl.pallas_call(
        body,
        grid=(_N // _BN,),
        in_specs=[
            pl.BlockSpec((_NC, _BN, _C), lambda i: (0, i, 0)),
            pl.BlockSpec((_BN, _C), lambda i: (i, 0)),
            pl.BlockSpec((_BN, _C), lambda i: (i, 0)),
            pl.BlockSpec((2 * _C, _C), lambda i: (0, 0)),
            pl.BlockSpec((1, _C), lambda i: (0, 0)),
        ],
        out_specs=pl.BlockSpec((_BN, 3 * _C), lambda i: (i, 0)),
        out_shape=jax.ShapeDtypeStruct((_N, 3 * _C), jnp.float32),
    )(sc3, h1, h2, root3, bias3)


def kernel(edge_index, edge_type, comp1, bases1, root1, bias1,
           comp2, bases2, root2, bias2, comp3, bases3, root3, bias3):
    ei = edge_index.astype(jnp.int32)
    et = edge_type.astype(jnp.int32)
    src = ei[0]
    dst = ei[1]

    g13, w = _sc_prologue(src, dst, et)
    w1 = _tc_table1(comp1, bases1).reshape(_R * _N, _C)
    sc1 = _sc_layer(w1, g13, dst, w)
    h1, xt2 = _tc_layer2(sc1, root1, bias1.reshape(1, _C), comp2, bases2)
    sc2 = _sc_layer(xt2.reshape(_R * _N, _C), g13, dst, w)
    h2, xt3 = _tc_layer3(sc2, h1, root2, bias2.reshape(1, _C), comp3, bases3)
    sc3 = _sc_layer(xt3.reshape(_R * _N, _C), g13, dst, w)
    return _tc_final(sc3, h1, h2, root3, bias3.reshape(1, _C))


# speedup vs baseline: 13.3243x; 1.2822x over previous
import functools

import jax
import jax.numpy as jnp
from jax import lax
from jax.experimental import pallas as pl
from jax.experimental.pallas import tpu as pltpu
from jax.experimental.pallas import tpu_sc as plsc

_N = 10000
_E = 320000
_R = 8
_NB = 4
_C = 128

_NC = 2
_NS = 16
_NW = _NC * _NS
_EPW = _E // _NW
_B = 80
_NBATCH = _EPW // _B
_EPT = _E // _NS
_NCB = _EPT // _B
_KPAD = 81920
_KSL = _KPAD // _NS

_BN = 1000


def _mesh():
    return plsc.VectorSubcoreMesh(core_axis_name="c", subcore_axis_name="s")


def _sc_prologue(src, dst, et):
    @functools.partial(
        pl.kernel,
        out_type=[jax.ShapeDtypeStruct((_E,), jnp.int32),
                  jax.ShapeDtypeStruct((_E,), jnp.float32)],
        mesh=_mesh(),
        scratch_types=[
            pltpu.VMEM((_EPT,), jnp.int32),
            pltpu.VMEM((_EPT,), jnp.int32),
            pltpu.VMEM((_EPW,), jnp.int32),
            pltpu.VMEM((_EPW,), jnp.int32),
            pltpu.VMEM((_EPW,), jnp.int32),
            pltpu.VMEM((_EPW,), jnp.int32),
            pltpu.VMEM((_EPW + 16,), jnp.float32),
            pltpu.VMEM((_B,), jnp.int32),
            pltpu.VMEM((_B,), jnp.int32),
            pltpu.VMEM((_B,), jnp.float32),
            pltpu.VMEM((_KSL,), jnp.float32),
            pltpu.VMEM_SHARED((_KPAD,), jnp.float32),
            pltpu.SemaphoreType.DMA,
            pltpu.SemaphoreType.DMA,
        ],
    )
    def kfn(src_h, dst_h, et_h, g13_o, w_o,
            dstc, typc, srcp, dstp, typp, g13b, wb,
            key0, key1, ones, sbuf, cnt_sh, k0, k1):
        c = lax.axis_index("c")
        s = lax.axis_index("s")
        wid = s * _NC + c

        def zfill(i, carry):
            sbuf[pl.ds(i * 16, 16)] = jnp.zeros((16,), jnp.float32)
            return carry
        lax.fori_loop(0, _KSL // 16, zfill, 0)
        for j in range(_B // 16):
            ones[pl.ds(j * 16, 16)] = jnp.ones((16,), jnp.float32)
        pltpu.sync_copy(sbuf, cnt_sh.at[pl.ds(s * _KSL, _KSL)])
        pltpu.sync_copy(dst_h.at[pl.ds(s * _EPT, _EPT)], dstc)
        pltpu.sync_copy(et_h.at[pl.ds(s * _EPT, _EPT)], typc)
        plsc.subcore_barrier()

        def ckeys(a, key):
            for q in range(_B // 16):
                d = dstc[pl.ds(a * _B + q * 16, 16)]
                t = typc[pl.ds(a * _B + q * 16, 16)]
                key[pl.ds(q * 16, 16)] = d * _R + t

        def cstart(key, sem):
            pltpu.async_copy(ones, cnt_sh.at[key], sem, add=True)

        def cwait(key, sem):
            pltpu.make_async_copy(ones, cnt_sh.at[key], sem).wait()

        ckeys(0, key0)
        cstart(key0, k0)
        ckeys(1, key1)
        cstart(key1, k1)

        def cbody(k, carry):
            a = k * 2
            cwait(key0, k0)
            ckeys(a, key0)
            cstart(key0, k0)
            cwait(key1, k1)
            ckeys(a + 1, key1)
            cstart(key1, k1)
            return carry
        lax.fori_loop(1, _NCB // 2, cbody, 0)
        cwait(key0, k0)
        cwait(key1, k1)
        plsc.subcore_barrier()

        pltpu.sync_copy(cnt_sh.at[pl.ds(s * _KSL, _KSL)], sbuf)

        def rbody(i, carry):
            x = sbuf[pl.ds(i * 16, 16)]
            sbuf[pl.ds(i * 16, 16)] = 1.0 / jnp.maximum(x, 1.0)
            return carry
        lax.fori_loop(0, _KSL // 16, rbody, 0)
        pltpu.sync_copy(sbuf, cnt_sh.at[pl.ds(s * _KSL, _KSL)])
        plsc.subcore_barrier()

        pltpu.sync_copy(src_h.at[pl.ds(wid * _EPW, _EPW)], srcp)
        pltpu.sync_copy(dst_h.at[pl.ds(wid * _EPW, _EPW)], dstp)
        pltpu.sync_copy(et_h.at[pl.ds(wid * _EPW, _EPW)], typp)

        def pkeys(a, key):
            for q in range(_B // 16):
                sj = srcp[pl.ds(a * _B + q * 16, 16)]
                dj = dstp[pl.ds(a * _B + q * 16, 16)]
                tj = typp[pl.ds(a * _B + q * 16, 16)]
                key[pl.ds(q * 16, 16)] = dj * _R + tj
                g13b[pl.ds(a * _B + q * 16, 16)] = tj * _N + sj

        def gstart(a, key, sem):
            pltpu.async_copy(cnt_sh.at[key], wb.at[pl.ds(a * _B, _B)], sem)

        def gwait(key, sem):
            pltpu.make_async_copy(cnt_sh.at[key], wb.at[pl.ds(0, _B)],
                                  sem).wait()

        pkeys(0, key0)
        gstart(0, key0, k0)
        pkeys(1, key1)
        gstart(1, key1, k1)

        def pbody(k, carry):
            a = k * 2
            gwait(key0, k0)
            pkeys(a, key0)
            gstart(a, key0, k0)
            gwait(key1, k1)
            pkeys(a + 1, key1)
            gstart(a + 1, key1, k1)
            return carry
        lax.fori_loop(1, 62, pbody, 0)
        gwait(key0, k0)
        pkeys(124, key0)
        gstart(124, key0, k0)
        gwait(key1, k1)
        gwait(key0, k0)
        pltpu.sync_copy(g13b, g13_o.at[pl.ds(wid * _EPW, _EPW)])
        pltpu.sync_copy(wb.at[pl.ds(0, _EPW)], w_o.at[pl.ds(wid * _EPW, _EPW)])

    return kfn(src, dst, et)


_LB = 40
_LNB = _EPW // _LB
_CH = 10
_NCHUNK = _LNB // _CH


def _sc_layer(table, g13, dst, w):
    @functools.partial(
        pl.kernel,
        out_type=jax.ShapeDtypeStruct((_NC, _N, _C), jnp.float32),
        mesh=_mesh(),
        scratch_types=[
            pltpu.VMEM((_CH * _LB,), jnp.int32),
            pltpu.VMEM((_CH * _LB,), jnp.int32),
            pltpu.VMEM((_CH * _LB + 16,), jnp.float32),
            pltpu.VMEM((_CH * _LB + 16,), jnp.float32),
            pltpu.VMEM((_CH * _LB,), jnp.int32),
            pltpu.VMEM((_CH * _LB,), jnp.int32),
            pltpu.VMEM((_LB,), jnp.int32),
            pltpu.VMEM((_LB,), jnp.int32),
            pltpu.VMEM((_LB, _C), jnp.float32),
            pltpu.VMEM((_LB, _C), jnp.float32),
            pltpu.VMEM_SHARED((_N, _C), jnp.float32),
            pltpu.SemaphoreType.DMA,
            pltpu.SemaphoreType.DMA,
            pltpu.SemaphoreType.DMA,
            pltpu.SemaphoreType.DMA,
            pltpu.SemaphoreType.DMA,
            pltpu.SemaphoreType.DMA,
        ],
    )
    def kfn(table_h, g13_h, dst_h, w_h, out_h,
            kidx0, kidx1, kw0, kw1, kdst0, kdst1, dstb0, dstb1,
            rows0, rows1, acc, k0, k1, g0, g1, s0, s1):
        c = lax.axis_index("c")
        s = lax.axis_index("s")
        wid = s * _NC + c
        row_base = s * 632
        ec = _CH * _LB
        ebase = wid * _EPW

        def kstart(ci, kidx, kw, kdst, sem):
            pltpu.async_copy(g13_h.at[pl.ds(ebase + ci * ec, ec)], kidx, sem)
            pltpu.async_copy(w_h.at[pl.ds(ebase + ci * ec, ec)],
                             kw.at[pl.ds(0, ec)], sem)
            pltpu.async_copy(dst_h.at[pl.ds(ebase + ci * ec, ec)], kdst, sem)

        def kwait(kidx, kw, kdst, sem):
            pltpu.make_async_copy(g13_h.at[pl.ds(0, ec)], kidx, sem).wait()
            pltpu.make_async_copy(w_h.at[pl.ds(0, ec)],
                                  kw.at[pl.ds(0, ec)], sem).wait()
            pltpu.make_async_copy(dst_h.at[pl.ds(0, ec)], kdst, sem).wait()

        def gstart(kidx, la, rows, sem):
            pltpu.async_copy(table_h.at[kidx.at[pl.ds(la * _LB, _LB)]],
                             rows, sem)

        def gwait(rows, sem):
            pltpu.make_async_copy(table_h.at[kidx0.at[pl.ds(0, _LB)]],
                                  rows, sem).wait()

        def sstart(kdst, la, dstb, rows, sem):
            for o in (0, 16, 24):
                dstb[pl.ds(o, 16)] = kdst[pl.ds(la * _LB + o, 16)]
            pltpu.async_copy(rows, acc.at[dstb], sem, add=True)

        def swait(rows, sem):
            pltpu.make_async_copy(rows, acc.at[dstb0], sem).wait()

        def scale(kw, la, rows):
            def sq(q, carry):
                wv = kw[pl.ds(la * _LB + q * 8, 16)]
                for i in range(8):
                    e = q * 8 + i
                    ws = wv[i]
                    for j in range(_C // 16):
                        rows[e, pl.ds(j * 16, 16)] = (
                            rows[e, pl.ds(j * 16, 16)] * ws)
                return carry
            lax.fori_loop(0, _LB // 8, sq, 0)

        kstart(0, kidx0, kw0, kdst0, k0)
        kstart(1, kidx1, kw1, kdst1, k1)

        def zfill(i, carry):
            for j in range(_C // 16):
                rows0[i, pl.ds(j * 16, 16)] = jnp.zeros((16,), jnp.float32)
            return carry
        lax.fori_loop(0, _LB, zfill, 0)
        nfull = lax.select(s == _NS - 1, 13, 15)

        def zcopy(k, carry):
            pltpu.sync_copy(rows0, acc.at[pl.ds(row_base + k * _LB, _LB), :])
            return carry
        lax.fori_loop(0, nfull, zcopy, 0)

        @pl.when(s != _NS - 1)
        def _ztail():
            pltpu.sync_copy(rows0.at[pl.ds(0, 32), :],
                            acc.at[pl.ds(row_base + 600, 32), :])
        plsc.subcore_barrier()

        kwait(kidx0, kw0, kdst0, k0)
        gstart(kidx0, 0, rows0, g0)
        gstart(kidx0, 1, rows1, g1)

        def chunk(ci, kidx, kw, kdst, kidy, kwy, kdsy, semx, semy):
            def step(t, carry):
                la = t * 2
                gwait(rows0, g0)
                scale(kw, la, rows0)
                sstart(kdst, la, dstb0, rows0, s0)
                gwait(rows1, g1)
                scale(kw, la + 1, rows1)
                sstart(kdst, la + 1, dstb1, rows1, s1)
                swait(rows0, s0)
                gstart(kidx, la + 2, rows0, g0)
                swait(rows1, s1)
                gstart(kidx, la + 3, rows1, g1)
                return carry
            lax.fori_loop(0, _CH // 2 - 1, step, 0)
            gwait(rows0, g0)
            scale(kw, _CH - 2, rows0)
            sstart(kdst, _CH - 2, dstb0, rows0, s0)
            gwait(rows1, g1)
            scale(kw, _CH - 1, rows1)
            sstart(kdst, _CH - 1, dstb1, rows1, s1)
            swait(rows0, s0)
            swait(rows1, s1)

            @pl.when(ci + 1 < _NCHUNK)
            def _():
                kwait(kidy, kwy, kdsy, semy)
                gstart(kidy, 0, rows0, g0)
                gstart(kidy, 1, rows1, g1)

            @pl.when(ci + 2 < _NCHUNK)
            def _():
                kstart(ci + 2, kidx, kw, kdst, semx)

        def pair(cp, carry):
            chunk(cp * 2, kidx0, kw0, kdst0, kidx1, kw1, kdst1, k0, k1)
            chunk(cp * 2 + 1, kidx1, kw1, kdst1, kidx0, kw0, kdst0, k1, k0)
            return carry
        lax.fori_loop(0, (_NCHUNK - 1) // 2, pair, 0)
        chunk(_NCHUNK - 1, kidx0, kw0, kdst0, kidx1, kw1, kdst1, k0, k1)
        plsc.subcore_barrier()

        @pl.when(s != _NS - 1)
        def _drain_full():
            pltpu.sync_copy(acc.at[pl.ds(row_base, 632), :],
                            out_h.at[c, pl.ds(row_base, 632), :])

        @pl.when(s == _NS - 1)
        def _drain_tail():
            pltpu.sync_copy(acc.at[pl.ds(15 * 632, 520), :],
                            out_h.at[c, pl.ds(15 * 632, 520), :])

    return kfn(table, g13, dst, w)


def _tc_table1(comp1, bases1):
    def body(cm_ref, bb_ref, o_ref):
        cm = cm_ref[...]
        for r in range(_R):
            acc = cm[r, 0] * bb_ref[0]
            for b in range(1, _NB):
                acc = acc + cm[r, b] * bb_ref[b]
            o_ref[r] = acc
    return pl.pallas_call(
        body,
        grid=(_N // _BN,),
        in_specs=[
            pl.BlockSpec((_R, _NB), lambda i: (0, 0)),
            pl.BlockSpec((_NB, _BN, _C), lambda i: (0, i, 0)),
        ],
        out_specs=pl.BlockSpec((_R, _BN, _C), lambda i: (0, i, 0)),
        out_shape=jax.ShapeDtypeStruct((_R, _N, _C), jnp.float32),
    )(comp1, bases1)


def _tc_layer2(sc1, root1, bias1, comp2, bases2):
    def body(sc_ref, rt_ref, bs_ref, cm_ref, bb_ref, h1_ref, xt_ref):
        h1 = jnp.maximum(sc_ref[0] + sc_ref[1] + rt_ref[...] + bs_ref[...], 0.0)
        h1_ref[...] = h1
        cm = cm_ref[...]
        for r in range(_R):
            wr = cm[r, 0] * bb_ref[0]
            for b in range(1, _NB):
                wr = wr + cm[r, b] * bb_ref[b]
            xt_ref[r] = jnp.dot(h1, wr, preferred_element_type=jnp.float32)
    return pl.pallas_call(
        body,
        grid=(_N // _BN,),
        in_specs=[
            pl.BlockSpec((_NC, _BN, _C), lambda i: (0, i, 0)),
            pl.BlockSpec((_BN, _C), lambda i: (i, 0)),
            pl.BlockSpec((1, _C), lambda i: (0, 0)),
            pl.BlockSpec((_R, _NB), lambda i: (0, 0)),
            pl.BlockSpec((_NB, _C, _C), lambda i: (0, 0, 0)),
        ],
        out_specs=[
            pl.BlockSpec((_BN, _C), lambda i: (i, 0)),
            pl.BlockSpec((_R, _BN, _C), lambda i: (0, i, 0)),
        ],
        out_shape=[
            jax.ShapeDtypeStruct((_N, _C), jnp.float32),
            jax.ShapeDtypeStruct((_R, _N, _C), jnp.float32),
        ],
    )(sc1, root1, bias1, comp2, bases2)


def _tc_layer3(sc2, h1, root2, bias2, comp3, bases3):
    def body(sc_ref, h1_ref, rt_ref, bs_ref, cm_ref, bb_ref, h2_ref, xt_ref):
        h1 = h1_ref[...]
        h2 = jnp.maximum(
            sc_ref[0] + sc_ref[1]
            + jnp.dot(h1, rt_ref[...], preferred_element_type=jnp.float32)
            + bs_ref[...], 0.0)
        h2_ref[...] = h2
        f2 = jnp.concatenate([h1, h2], axis=-1)
        cm = cm_ref[...]
        for r in range(_R):
            wr = cm[r, 0] * bb_ref[0]
            for b in range(1, _NB):
                wr = wr + cm[r, b] * bb_ref[b]
            xt_ref[r] = jnp.dot(f2, wr, preferred_element_type=jnp.float32)
    return pl.pallas_call(
        body,
        grid=(_N // _BN,),
        in_specs=[
            pl.BlockSpec((_NC, _BN, _C), lambda i: (0, i, 0)),
            pl.BlockSpec((_BN, _C), lambda i: (i, 0)),
            pl.BlockSpec((_C, _C), lambda i: (0, 0)),
            pl.BlockSpec((1, _C), lambda i: (0, 0)),
            pl.BlockSpec((_R, _NB), lambda i: (0, 0)),
            pl.BlockSpec((_NB, 2 * _C, _C), lambda i: (0, 0, 0)),
        ],
        out_specs=[
            pl.BlockSpec((_BN, _C), lambda i: (i, 0)),
            pl.BlockSpec((_R, _BN, _C), lambda i: (0, i, 0)),
        ],
        out_shape=[
            jax.ShapeDtypeStruct((_N, _C), jnp.float32),
            jax.ShapeDtypeStruct((_R, _N, _C), jnp.float32),
        ],
    )(sc2, h1, root2, bias2, comp3, bases3)


def _tc_final(sc3, h1, h2, root3, bias3):
    def body(sc_ref, h1_ref, h2_ref, rt_ref, bs_ref, o_ref):
        h1 = h1_ref[...]
        h2 = h2_ref[...]
        rt = rt_ref[...]
        h3 = jnp.maximum(
            sc_ref[0] + sc_ref[1]
            + jnp.dot(h1, rt[:_C], preferred_element_type=jnp.float32)
            + jnp.dot(h2, rt[_C:], preferred_element_type=jnp.float32)
            + bs_ref[...], 0.0)
        f3 = jnp.concatenate([h1, h2, h3], axis=-1)
        m = jnp.max(f3, axis=-1, keepdims=True)
        lse = jnp.log(jnp.sum(jnp.exp(f3 - m), axis=-1, keepdims=True)) + m
        o_ref[...] = f3 - lse
    return pl.pallas_call(
        body,
        grid=(_N // _BN,),
        in_specs=[
            pl.BlockSpec((_NC, _BN, _C), lambda i: (0, i, 0)),
            pl.BlockSpec((_BN, _C), lambda i: (i, 0)),
            pl.BlockSpec((_BN, _C), lambda i: (i, 0)),
            pl.BlockSpec((2 * _C, _C), lambda i: (0, 0)),
            pl.BlockSpec((1, _C), lambda i: (0, 0)),
        ],
        out_specs=pl.BlockSpec((_BN, 3 * _C), lambda i: (i, 0)),
        out_shape=jax.ShapeDtypeStruct((_N, 3 * _C), jnp.float32),
    )(sc3, h1, h2, root3, bias3)


def kernel(edge_index, edge_type, comp1, bases1, root1, bias1,
           comp2, bases2, root2, bias2, comp3, bases3, root3, bias3):
    ei = edge_index.astype(jnp.int32)
    et = edge_type.astype(jnp.int32)
    src = ei[0]
    dst = ei[1]

    g13, w = _sc_prologue(src, dst, et)
    w1 = _tc_table1(comp1, bases1).reshape(_R * _N, _C)
    sc1 = _sc_layer(w1, g13, dst, w)
    h1, xt2 = _tc_layer2(sc1, root1, bias1.reshape(1, _C), comp2, bases2)
    sc2 = _sc_layer(xt2.reshape(_R * _N, _C), g13, dst, w)
    h2, xt3 = _tc_layer3(sc2, h1, root2, bias2.reshape(1, _C), comp3, bases3)
    sc3 = _sc_layer(xt3.reshape(_R * _N, _C), g13, dst, w)
    return _tc_final(sc3, h1, h2, root3, bias3.reshape(1, _C))


# speedup vs baseline: 17.1538x; 1.2874x over previous
import functools

import jax
import jax.numpy as jnp
from jax import lax
from jax.experimental import pallas as pl
from jax.experimental.pallas import tpu as pltpu
from jax.experimental.pallas import tpu_sc as plsc

_N = 10000
_E = 320000
_R = 8
_NB = 4
_C = 128

_NC = 2
_NS = 16
_NW = _NC * _NS
_EPW = _E // _NW
_B = 80
_NBATCH = _EPW // _B
_EPT = _E // _NS
_NCB = _EPT // _B
_KPAD = 81920
_KSL = _KPAD // _NS

_BN = 1000


def _mesh():
    return plsc.VectorSubcoreMesh(core_axis_name="c", subcore_axis_name="s")


def _sc_prologue(src, dst, et):
    @functools.partial(
        pl.kernel,
        out_type=[jax.ShapeDtypeStruct((_E,), jnp.int32),
                  jax.ShapeDtypeStruct((_E,), jnp.float32)],
        mesh=_mesh(),
        scratch_types=[
            pltpu.VMEM((_EPT,), jnp.int32),
            pltpu.VMEM((_EPT,), jnp.int32),
            pltpu.VMEM((_EPW,), jnp.int32),
            pltpu.VMEM((_EPW,), jnp.int32),
            pltpu.VMEM((_EPW,), jnp.int32),
            pltpu.VMEM((_EPW,), jnp.int32),
            pltpu.VMEM((_EPW + 16,), jnp.float32),
            pltpu.VMEM((_B,), jnp.int32),
            pltpu.VMEM((_B,), jnp.int32),
            pltpu.VMEM((_B,), jnp.float32),
            pltpu.VMEM((_KSL,), jnp.float32),
            pltpu.VMEM_SHARED((_KPAD,), jnp.float32),
            pltpu.SemaphoreType.DMA,
            pltpu.SemaphoreType.DMA,
        ],
    )
    def kfn(src_h, dst_h, et_h, g13_o, w_o,
            dstc, typc, srcp, dstp, typp, g13b, wb,
            key0, key1, ones, sbuf, cnt_sh, k0, k1):
        c = lax.axis_index("c")
        s = lax.axis_index("s")
        wid = s * _NC + c

        def zfill(i, carry):
            sbuf[pl.ds(i * 16, 16)] = jnp.zeros((16,), jnp.float32)
            return carry
        lax.fori_loop(0, _KSL // 16, zfill, 0)
        for j in range(_B // 16):
            ones[pl.ds(j * 16, 16)] = jnp.ones((16,), jnp.float32)
        pltpu.sync_copy(sbuf, cnt_sh.at[pl.ds(s * _KSL, _KSL)])
        pltpu.sync_copy(dst_h.at[pl.ds(s * _EPT, _EPT)], dstc)
        pltpu.sync_copy(et_h.at[pl.ds(s * _EPT, _EPT)], typc)
        plsc.subcore_barrier()

        def ckeys(a, key):
            for q in range(_B // 16):
                d = dstc[pl.ds(a * _B + q * 16, 16)]
                t = typc[pl.ds(a * _B + q * 16, 16)]
                key[pl.ds(q * 16, 16)] = d * _R + t

        def cstart(key, sem):
            pltpu.async_copy(ones, cnt_sh.at[key], sem, add=True)

        def cwait(key, sem):
            pltpu.make_async_copy(ones, cnt_sh.at[key], sem).wait()

        ckeys(0, key0)
        cstart(key0, k0)
        ckeys(1, key1)
        cstart(key1, k1)

        def cbody(k, carry):
            a = k * 2
            cwait(key0, k0)
            ckeys(a, key0)
            cstart(key0, k0)
            cwait(key1, k1)
            ckeys(a + 1, key1)
            cstart(key1, k1)
            return carry
        lax.fori_loop(1, _NCB // 2, cbody, 0)
        cwait(key0, k0)
        cwait(key1, k1)
        plsc.subcore_barrier()

        pltpu.sync_copy(cnt_sh.at[pl.ds(s * _KSL, _KSL)], sbuf)

        def rbody(i, carry):
            x = sbuf[pl.ds(i * 16, 16)]
            sbuf[pl.ds(i * 16, 16)] = 1.0 / jnp.maximum(x, 1.0)
            return carry
        lax.fori_loop(0, _KSL // 16, rbody, 0)
        pltpu.sync_copy(sbuf, cnt_sh.at[pl.ds(s * _KSL, _KSL)])
        plsc.subcore_barrier()

        pltpu.sync_copy(src_h.at[pl.ds(wid * _EPW, _EPW)], srcp)
        pltpu.sync_copy(dst_h.at[pl.ds(wid * _EPW, _EPW)], dstp)
        pltpu.sync_copy(et_h.at[pl.ds(wid * _EPW, _EPW)], typp)

        def pkeys(a, key):
            for q in range(_B // 16):
                sj = srcp[pl.ds(a * _B + q * 16, 16)]
                dj = dstp[pl.ds(a * _B + q * 16, 16)]
                tj = typp[pl.ds(a * _B + q * 16, 16)]
                key[pl.ds(q * 16, 16)] = dj * _R + tj
                g13b[pl.ds(a * _B + q * 16, 16)] = tj * _N + sj

        def gstart(a, key, sem):
            pltpu.async_copy(cnt_sh.at[key], wb.at[pl.ds(a * _B, _B)], sem)

        def gwait(key, sem):
            pltpu.make_async_copy(cnt_sh.at[key], wb.at[pl.ds(0, _B)],
                                  sem).wait()

        pkeys(0, key0)
        gstart(0, key0, k0)
        pkeys(1, key1)
        gstart(1, key1, k1)

        def pbody(k, carry):
            a = k * 2
            gwait(key0, k0)
            pkeys(a, key0)
            gstart(a, key0, k0)
            gwait(key1, k1)
            pkeys(a + 1, key1)
            gstart(a + 1, key1, k1)
            return carry
        lax.fori_loop(1, 62, pbody, 0)
        gwait(key0, k0)
        pkeys(124, key0)
        gstart(124, key0, k0)
        gwait(key1, k1)
        gwait(key0, k0)
        pltpu.sync_copy(g13b, g13_o.at[pl.ds(wid * _EPW, _EPW)])
        pltpu.sync_copy(wb.at[pl.ds(0, _EPW)], w_o.at[pl.ds(wid * _EPW, _EPW)])

    return kfn(src, dst, et)


_LB = 40
_LNB = _EPW // _LB
_CH = 10
_NCHUNK = _LNB // _CH


def _sc_layer(table, g13, dst, w):
    @functools.partial(
        pl.kernel,
        out_type=jax.ShapeDtypeStruct((_NC, _N, _C), jnp.float32),
        mesh=_mesh(),
        scratch_types=[
            pltpu.VMEM((_EPW,), jnp.int32),
            pltpu.VMEM((_EPW + 16,), jnp.float32),
            pltpu.VMEM((_EPW,), jnp.int32),
            pltpu.VMEM((_LB,), jnp.int32),
            pltpu.VMEM((_LB,), jnp.int32),
            pltpu.VMEM((_LB,), jnp.int32),
            pltpu.VMEM((_LB, _C), jnp.float32),
            pltpu.VMEM((_LB, _C), jnp.float32),
            pltpu.VMEM((_LB, _C), jnp.float32),
            pltpu.VMEM_SHARED((_N, _C), jnp.float32),
            pltpu.SemaphoreType.DMA,
            pltpu.SemaphoreType.DMA,
            pltpu.SemaphoreType.DMA,
            pltpu.SemaphoreType.DMA,
            pltpu.SemaphoreType.DMA,
            pltpu.SemaphoreType.DMA,
        ],
    )
    def kfn(table_h, g13_h, dst_h, w_h, out_h,
            gidx_all, w_all, dst_all, dstb0, dstb1, dstb2,
            rows0, rows1, rows2, acc, g0, g1, g2, s0, s1, s2):
        c = lax.axis_index("c")
        s = lax.axis_index("s")
        wid = s * _NC + c
        row_base = s * 632
        ebase = wid * _EPW
        rows = (rows0, rows1, rows2)
        dstb = (dstb0, dstb1, dstb2)
        gsem = (g0, g1, g2)
        ssem = (s0, s1, s2)

        pltpu.sync_copy(g13_h.at[pl.ds(ebase, _EPW)], gidx_all)
        pltpu.sync_copy(w_h.at[pl.ds(ebase, _EPW)], w_all.at[pl.ds(0, _EPW)])
        pltpu.sync_copy(dst_h.at[pl.ds(ebase, _EPW)], dst_all)

        def zfill(i, carry):
            for j in range(_C // 16):
                rows0[i, pl.ds(j * 16, 16)] = jnp.zeros((16,), jnp.float32)
            return carry
        lax.fori_loop(0, _LB, zfill, 0)
        nfull = lax.select(s == _NS - 1, 13, 15)

        def zcopy(k, carry):
            pltpu.sync_copy(rows0, acc.at[pl.ds(row_base + k * _LB, _LB), :])
            return carry
        lax.fori_loop(0, nfull, zcopy, 0)

        @pl.when(s != _NS - 1)
        def _ztail():
            pltpu.sync_copy(rows0.at[pl.ds(0, 32), :],
                            acc.at[pl.ds(row_base + 600, 32), :])
        plsc.subcore_barrier()

        def gstart(ib, u):
            pltpu.async_copy(
                table_h.at[gidx_all.at[pl.ds(ib * _LB, _LB)]], rows[u],
                gsem[u])

        def gwait(u):
            pltpu.make_async_copy(table_h.at[gidx_all.at[pl.ds(0, _LB)]],
                                  rows[u], gsem[u]).wait()

        def sstart(ib, u):
            for o in (0, 16, 24):
                dstb[u][pl.ds(o, 16)] = dst_all[pl.ds(ib * _LB + o, 16)]
            pltpu.async_copy(rows[u], acc.at[dstb[u]], ssem[u], add=True)

        def swait(u):
            pltpu.make_async_copy(rows[u], acc.at[dstb0], ssem[u]).wait()

        def scale(ib, u):
            def sq(q, carry):
                wv = w_all[pl.ds(ib * _LB + q * 8, 16)]
                for i in range(8):
                    e = q * 8 + i
                    ws = wv[i]
                    for j in range(_C // 16):
                        rows[u][e, pl.ds(j * 16, 16)] = (
                            rows[u][e, pl.ds(j * 16, 16)] * ws)
                return carry
            lax.fori_loop(0, _LB // 8, sq, 0)

        gstart(0, 0)
        gstart(1, 1)

        def body(k, carry):
            for u in range(3):
                ib = k * 3 + u
                z = (u + 2) % 3
                gwait(u)
                scale(ib, u)
                sstart(ib, u)

                @pl.when(ib >= 1)
                def _():
                    swait(z)

                @pl.when(ib + 2 < _LNB)
                def _():
                    gstart(ib + 2, z)
            return carry
        lax.fori_loop(0, (_LNB - 1) // 3, body, 0)
        gwait(0)
        scale(_LNB - 1, 0)
        sstart(_LNB - 1, 0)
        swait(2)
        swait(0)
        plsc.subcore_barrier()

        @pl.when(s != _NS - 1)
        def _drain_full():
            pltpu.sync_copy(acc.at[pl.ds(row_base, 632), :],
                            out_h.at[c, pl.ds(row_base, 632), :])

        @pl.when(s == _NS - 1)
        def _drain_tail():
            pltpu.sync_copy(acc.at[pl.ds(15 * 632, 520), :],
                            out_h.at[c, pl.ds(15 * 632, 520), :])

    return kfn(table, g13, dst, w)


def _tc_table1(comp1, bases1):
    def body(cm_ref, bb_ref, o_ref):
        cm = cm_ref[...]
        for r in range(_R):
            acc = cm[r, 0] * bb_ref[0]
            for b in range(1, _NB):
                acc = acc + cm[r, b] * bb_ref[b]
            o_ref[r] = acc
    return pl.pallas_call(
        body,
        grid=(_N // _BN,),
        in_specs=[
            pl.BlockSpec((_R, _NB), lambda i: (0, 0)),
            pl.BlockSpec((_NB, _BN, _C), lambda i: (0, i, 0)),
        ],
        out_specs=pl.BlockSpec((_R, _BN, _C), lambda i: (0, i, 0)),
        out_shape=jax.ShapeDtypeStruct((_R, _N, _C), jnp.float32),
    )(comp1, bases1)


def _tc_layer2(sc1, root1, bias1, comp2, bases2):
    def body(sc_ref, rt_ref, bs_ref, cm_ref, bb_ref, h1_ref, xt_ref):
        h1 = jnp.maximum(sc_ref[0] + sc_ref[1] + rt_ref[...] + bs_ref[...], 0.0)
        h1_ref[...] = h1
        cm = cm_ref[...]
        for r in range(_R):
            wr = cm[r, 0] * bb_ref[0]
            for b in range(1, _NB):
                wr = wr + cm[r, b] * bb_ref[b]
            xt_ref[r] = jnp.dot(h1, wr, preferred_element_type=jnp.float32)
    return pl.pallas_call(
        body,
        grid=(_N // _BN,),
        in_specs=[
            pl.BlockSpec((_NC, _BN, _C), lambda i: (0, i, 0)),
            pl.BlockSpec((_BN, _C), lambda i: (i, 0)),
            pl.BlockSpec((1, _C), lambda i: (0, 0)),
            pl.BlockSpec((_R, _NB), lambda i: (0, 0)),
            pl.BlockSpec((_NB, _C, _C), lambda i: (0, 0, 0)),
        ],
        out_specs=[
            pl.BlockSpec((_BN, _C), lambda i: (i, 0)),
            pl.BlockSpec((_R, _BN, _C), lambda i: (0, i, 0)),
        ],
        out_shape=[
            jax.ShapeDtypeStruct((_N, _C), jnp.float32),
            jax.ShapeDtypeStruct((_R, _N, _C), jnp.float32),
        ],
    )(sc1, root1, bias1, comp2, bases2)


def _tc_layer3(sc2, h1, root2, bias2, comp3, bases3):
    def body(sc_ref, h1_ref, rt_ref, bs_ref, cm_ref, bb_ref, h2_ref, xt_ref):
        h1 = h1_ref[...]
        h2 = jnp.maximum(
            sc_ref[0] + sc_ref[1]
            + jnp.dot(h1, rt_ref[...], preferred_element_type=jnp.float32)
            + bs_ref[...], 0.0)
        h2_ref[...] = h2
        f2 = jnp.concatenate([h1, h2], axis=-1)
        cm = cm_ref[...]
        for r in range(_R):
            wr = cm[r, 0] * bb_ref[0]
            for b in range(1, _NB):
                wr = wr + cm[r, b] * bb_ref[b]
            xt_ref[r] = jnp.dot(f2, wr, preferred_element_type=jnp.float32)
    return pl.pallas_call(
        body,
        grid=(_N // _BN,),
        in_specs=[
            pl.BlockSpec((_NC, _BN, _C), lambda i: (0, i, 0)),
            pl.BlockSpec((_BN, _C), lambda i: (i, 0)),
            pl.BlockSpec((_C, _C), lambda i: (0, 0)),
            pl.BlockSpec((1, _C), lambda i: (0, 0)),
            pl.BlockSpec((_R, _NB), lambda i: (0, 0)),
            pl.BlockSpec((_NB, 2 * _C, _C), lambda i: (0, 0, 0)),
        ],
        out_specs=[
            pl.BlockSpec((_BN, _C), lambda i: (i, 0)),
            pl.BlockSpec((_R, _BN, _C), lambda i: (0, i, 0)),
        ],
        out_shape=[
            jax.ShapeDtypeStruct((_N, _C), jnp.float32),
            jax.ShapeDtypeStruct((_R, _N, _C), jnp.float32),
        ],
    )(sc2, h1, root2, bias2, comp3, bases3)


def _tc_final(sc3, h1, h2, root3, bias3):
    def body(sc_ref, h1_ref, h2_ref, rt_ref, bs_ref, o_ref):
        h1 = h1_ref[...]
        h2 = h2_ref[...]
        rt = rt_ref[...]
        h3 = jnp.maximum(
            sc_ref[0] + sc_ref[1]
            + jnp.dot(h1, rt[:_C], preferred_element_type=jnp.float32)
            + jnp.dot(h2, rt[_C:], preferred_element_type=jnp.float32)
            + bs_ref[...], 0.0)
        f3 = jnp.concatenate([h1, h2, h3], axis=-1)
        m = jnp.max(f3, axis=-1, keepdims=True)
        lse = jnp.log(jnp.sum(jnp.exp(f3 - m), axis=-1, keepdims=True)) + m
        o_ref[...] = f3 - lse
    return pl.pallas_call(
        body,
        grid=(_N // _BN,),
        in_specs=[
            pl.BlockSpec((_NC, _BN, _C), lambda i: (0, i, 0)),
            pl.BlockSpec((_BN, _C), lambda i: (i, 0)),
            pl.BlockSpec((_BN, _C), lambda i: (i, 0)),
            pl.BlockSpec((2 * _C, _C), lambda i: (0, 0)),
            pl.BlockSpec((1, _C), lambda i: (0, 0)),
        ],
        out_specs=pl.BlockSpec((_BN, 3 * _C), lambda i: (i, 0)),
        out_shape=jax.ShapeDtypeStruct((_N, 3 * _C), jnp.float32),
    )(sc3, h1, h2, root3, bias3)


def kernel(edge_index, edge_type, comp1, bases1, root1, bias1,
           comp2, bases2, root2, bias2, comp3, bases3, root3, bias3):
    ei = edge_index.astype(jnp.int32)
    et = edge_type.astype(jnp.int32)
    src = ei[0]
    dst = ei[1]

    g13, w = _sc_prologue(src, dst, et)
    w1 = _tc_table1(comp1, bases1).reshape(_R * _N, _C)
    sc1 = _sc_layer(w1, g13, dst, w)
    h1, xt2 = _tc_layer2(sc1, root1, bias1.reshape(1, _C), comp2, bases2)
    sc2 = _sc_layer(xt2.reshape(_R * _N, _C), g13, dst, w)
    h2, xt3 = _tc_layer3(sc2, h1, root2, bias2.reshape(1, _C), comp3, bases3)
    sc3 = _sc_layer(xt3.reshape(_R * _N, _C), g13, dst, w)
    return _tc_final(sc3, h1, h2, root3, bias3.reshape(1, _C))


# speedup vs baseline: 32.9091x; 1.9185x over previous
import functools

import jax
import jax.numpy as jnp
from jax import lax
from jax.experimental import pallas as pl
from jax.experimental.pallas import tpu as pltpu
from jax.experimental.pallas import tpu_sc as plsc

_N = 10000
_E = 320000
_R = 8
_NB = 4
_C = 128

_NC = 2
_NS = 16
_NW = _NC * _NS
_EPW = _E // _NW
_B = 80
_NBATCH = _EPW // _B
_EPT = _E // _NS
_NCB = _EPT // _B
_KPAD = 81920
_KSL = _KPAD // _NS

_BN = 1000


def _mesh():
    return plsc.VectorSubcoreMesh(core_axis_name="c", subcore_axis_name="s")


def _sc_prologue(src, dst, et):
    @functools.partial(
        pl.kernel,
        out_type=[jax.ShapeDtypeStruct((_E,), jnp.int32),
                  jax.ShapeDtypeStruct((_E,), jnp.float32)],
        mesh=_mesh(),
        scratch_types=[
            pltpu.VMEM((_EPT,), jnp.int32),
            pltpu.VMEM((_EPT,), jnp.int32),
            pltpu.VMEM((_EPW,), jnp.int32),
            pltpu.VMEM((_EPW,), jnp.int32),
            pltpu.VMEM((_EPW,), jnp.int32),
            pltpu.VMEM((_EPW,), jnp.int32),
            pltpu.VMEM((_EPW + 16,), jnp.float32),
            pltpu.VMEM((_B,), jnp.int32),
            pltpu.VMEM((_B,), jnp.int32),
            pltpu.VMEM((_B,), jnp.float32),
            pltpu.VMEM((_KSL,), jnp.float32),
            pltpu.VMEM_SHARED((_KPAD,), jnp.float32),
            pltpu.SemaphoreType.DMA,
            pltpu.SemaphoreType.DMA,
        ],
    )
    def kfn(src_h, dst_h, et_h, g13_o, w_o,
            dstc, typc, srcp, dstp, typp, g13b, wb,
            key0, key1, ones, sbuf, cnt_sh, k0, k1):
        c = lax.axis_index("c")
        s = lax.axis_index("s")
        wid = s * _NC + c

        def zfill(i, carry):
            sbuf[pl.ds(i * 16, 16)] = jnp.zeros((16,), jnp.float32)
            return carry
        lax.fori_loop(0, _KSL // 16, zfill, 0)
        for j in range(_B // 16):
            ones[pl.ds(j * 16, 16)] = jnp.ones((16,), jnp.float32)
        pltpu.sync_copy(sbuf, cnt_sh.at[pl.ds(s * _KSL, _KSL)])
        pltpu.sync_copy(dst_h.at[pl.ds(s * _EPT, _EPT)], dstc)
        pltpu.sync_copy(et_h.at[pl.ds(s * _EPT, _EPT)], typc)
        plsc.subcore_barrier()

        def ckeys(a, key):
            for q in range(_B // 16):
                d = dstc[pl.ds(a * _B + q * 16, 16)]
                t = typc[pl.ds(a * _B + q * 16, 16)]
                key[pl.ds(q * 16, 16)] = d * _R + t

        def cstart(key, sem):
            pltpu.async_copy(ones, cnt_sh.at[key], sem, add=True)

        def cwait(key, sem):
            pltpu.make_async_copy(ones, cnt_sh.at[key], sem).wait()

        ckeys(0, key0)
        cstart(key0, k0)
        ckeys(1, key1)
        cstart(key1, k1)

        def cbody(k, carry):
            a = k * 2
            cwait(key0, k0)
            ckeys(a, key0)
            cstart(key0, k0)
            cwait(key1, k1)
            ckeys(a + 1, key1)
            cstart(key1, k1)
            return carry
        lax.fori_loop(1, _NCB // 2, cbody, 0)
        cwait(key0, k0)
        cwait(key1, k1)
        plsc.subcore_barrier()

        pltpu.sync_copy(cnt_sh.at[pl.ds(s * _KSL, _KSL)], sbuf)

        def rbody(i, carry):
            x = sbuf[pl.ds(i * 16, 16)]
            sbuf[pl.ds(i * 16, 16)] = 1.0 / jnp.maximum(x, 1.0)
            return carry
        lax.fori_loop(0, _KSL // 16, rbody, 0)
        pltpu.sync_copy(sbuf, cnt_sh.at[pl.ds(s * _KSL, _KSL)])
        plsc.subcore_barrier()

        pltpu.sync_copy(src_h.at[pl.ds(wid * _EPW, _EPW)], srcp)
        pltpu.sync_copy(dst_h.at[pl.ds(wid * _EPW, _EPW)], dstp)
        pltpu.sync_copy(et_h.at[pl.ds(wid * _EPW, _EPW)], typp)

        def pkeys(a, key):
            for q in range(_B // 16):
                sj = srcp[pl.ds(a * _B + q * 16, 16)]
                dj = dstp[pl.ds(a * _B + q * 16, 16)]
                tj = typp[pl.ds(a * _B + q * 16, 16)]
                key[pl.ds(q * 16, 16)] = dj * _R + tj
                g13b[pl.ds(a * _B + q * 16, 16)] = tj * _N + sj

        def gstart(a, key, sem):
            pltpu.async_copy(cnt_sh.at[key], wb.at[pl.ds(a * _B, _B)], sem)

        def gwait(key, sem):
            pltpu.make_async_copy(cnt_sh.at[key], wb.at[pl.ds(0, _B)],
                                  sem).wait()

        pkeys(0, key0)
        gstart(0, key0, k0)
        pkeys(1, key1)
        gstart(1, key1, k1)

        def pbody(k, carry):
            a = k * 2
            gwait(key0, k0)
            pkeys(a, key0)
            gstart(a, key0, k0)
            gwait(key1, k1)
            pkeys(a + 1, key1)
            gstart(a + 1, key1, k1)
            return carry
        lax.fori_loop(1, 62, pbody, 0)
        gwait(key0, k0)
        pkeys(124, key0)
        gstart(124, key0, k0)
        gwait(key1, k1)
        gwait(key0, k0)
        pltpu.sync_copy(g13b, g13_o.at[pl.ds(wid * _EPW, _EPW)])
        pltpu.sync_copy(wb.at[pl.ds(0, _EPW)], w_o.at[pl.ds(wid * _EPW, _EPW)])

    return kfn(src, dst, et)


_LB = 40
_LNB = _EPW // _LB
_CH = 10
_NCHUNK = _LNB // _CH


def _sc_layer(table, g13, dst, w):
    @functools.partial(
        pl.kernel,
        out_type=jax.ShapeDtypeStruct((_NC, _N, _C), jnp.float32),
        mesh=_mesh(),
        scratch_types=[
            pltpu.VMEM((_EPW,), jnp.int32),
            pltpu.VMEM((_EPW + 16,), jnp.float32),
            pltpu.VMEM((_EPW,), jnp.int32),
            pltpu.VMEM((_LB,), jnp.int32),
            pltpu.VMEM((_LB,), jnp.int32),
            pltpu.VMEM((_LB,), jnp.int32),
            pltpu.VMEM((_LB, _C), jnp.float32),
            pltpu.VMEM((_LB, _C), jnp.float32),
            pltpu.VMEM((_LB, _C), jnp.float32),
            pltpu.VMEM_SHARED((_N, _C), jnp.float32),
            pltpu.SemaphoreType.DMA,
            pltpu.SemaphoreType.DMA,
            pltpu.SemaphoreType.DMA,
            pltpu.SemaphoreType.DMA,
            pltpu.SemaphoreType.DMA,
            pltpu.SemaphoreType.DMA,
        ],
    )
    def kfn(table_h, g13_h, dst_h, w_h, out_h,
            gidx_all, w_all, dst_all, dstb0, dstb1, dstb2,
            rows0, rows1, rows2, acc, g0, g1, g2, s0, s1, s2):
        c = lax.axis_index("c")
        s = lax.axis_index("s")
        wid = s * _NC + c
        row_base = s * 632
        ebase = wid * _EPW
        rows = (rows0, rows1, rows2)
        dstb = (dstb0, dstb1, dstb2)
        gsem = (g0, g1, g2)
        ssem = (s0, s1, s2)

        pltpu.sync_copy(g13_h.at[pl.ds(ebase, _EPW)], gidx_all)
        pltpu.sync_copy(w_h.at[pl.ds(ebase, _EPW)], w_all.at[pl.ds(0, _EPW)])
        pltpu.sync_copy(dst_h.at[pl.ds(ebase, _EPW)], dst_all)

        def zfill(i, carry):
            for j in range(_C // 16):
                rows0[i, pl.ds(j * 16, 16)] = jnp.zeros((16,), jnp.float32)
            return carry
        lax.fori_loop(0, _LB, zfill, 0)
        nfull = lax.select(s == _NS - 1, 13, 15)

        def zcopy(k, carry):
            pltpu.sync_copy(rows0, acc.at[pl.ds(row_base + k * _LB, _LB), :])
            return carry
        lax.fori_loop(0, nfull, zcopy, 0)

        @pl.when(s != _NS - 1)
        def _ztail():
            pltpu.sync_copy(rows0.at[pl.ds(0, 32), :],
                            acc.at[pl.ds(row_base + 600, 32), :])
        plsc.subcore_barrier()

        def gstart(ib, u):
            pltpu.async_copy(
                table_h.at[gidx_all.at[pl.ds(ib * _LB, _LB)]], rows[u],
                gsem[u])

        def gwait(u):
            pltpu.make_async_copy(table_h.at[gidx_all.at[pl.ds(0, _LB)]],
                                  rows[u], gsem[u]).wait()

        def sstart(ib, u):
            for o in (0, 16, 24):
                dstb[u][pl.ds(o, 16)] = dst_all[pl.ds(ib * _LB + o, 16)]
            pltpu.async_copy(rows[u], acc.at[dstb[u]], ssem[u], add=True)

        def swait(u):
            pltpu.make_async_copy(rows[u], acc.at[dstb0], ssem[u]).wait()

        def scale(ib, u):
            for q in range(_LB // 16):
                wv = w_all[pl.ds(ib * _LB + q * 16, 16)]
                for i in range(16):
                    e = q * 16 + i
                    ws = wv[i]
                    for j in range(_C // 16):
                        rows[u][e, pl.ds(j * 16, 16)] = (
                            rows[u][e, pl.ds(j * 16, 16)] * ws)
            wv = w_all[pl.ds(ib * _LB + 32, 16)]
            for i in range(8):
                e = 32 + i
                ws = wv[i]
                for j in range(_C // 16):
                    rows[u][e, pl.ds(j * 16, 16)] = (
                        rows[u][e, pl.ds(j * 16, 16)] * ws)

        gstart(0, 0)
        gstart(1, 1)

        def body(k, carry):
            for u in range(3):
                ib = k * 3 + u
                z = (u + 2) % 3
                gwait(u)
                scale(ib, u)
                sstart(ib, u)

                @pl.when(ib >= 1)
                def _():
                    swait(z)

                @pl.when(ib + 2 < _LNB)
                def _():
                    gstart(ib + 2, z)
            return carry
        lax.fori_loop(0, (_LNB - 1) // 3, body, 0)
        gwait(0)
        scale(_LNB - 1, 0)
        sstart(_LNB - 1, 0)
        swait(2)
        swait(0)
        plsc.subcore_barrier()

        @pl.when(s != _NS - 1)
        def _drain_full():
            pltpu.sync_copy(acc.at[pl.ds(row_base, 632), :],
                            out_h.at[c, pl.ds(row_base, 632), :])

        @pl.when(s == _NS - 1)
        def _drain_tail():
            pltpu.sync_copy(acc.at[pl.ds(15 * 632, 520), :],
                            out_h.at[c, pl.ds(15 * 632, 520), :])

    return kfn(table, g13, dst, w)


def _tc_table1(comp1, bases1):
    def body(cm_ref, bb_ref, o_ref):
        cm = cm_ref[...]
        for r in range(_R):
            acc = cm[r, 0] * bb_ref[0]
            for b in range(1, _NB):
                acc = acc + cm[r, b] * bb_ref[b]
            o_ref[r] = acc
    return pl.pallas_call(
        body,
        grid=(_N // _BN,),
        in_specs=[
            pl.BlockSpec((_R, _NB), lambda i: (0, 0)),
            pl.BlockSpec((_NB, _BN, _C), lambda i: (0, i, 0)),
        ],
        out_specs=pl.BlockSpec((_R, _BN, _C), lambda i: (0, i, 0)),
        out_shape=jax.ShapeDtypeStruct((_R, _N, _C), jnp.float32),
    )(comp1, bases1)


def _tc_layer2(sc1, root1, bias1, comp2, bases2):
    def body(sc_ref, rt_ref, bs_ref, cm_ref, bb_ref, h1_ref, xt_ref):
        h1 = jnp.maximum(sc_ref[0] + sc_ref[1] + rt_ref[...] + bs_ref[...], 0.0)
        h1_ref[...] = h1
        cm = cm_ref[...]
        for r in range(_R):
            wr = cm[r, 0] * bb_ref[0]
            for b in range(1, _NB):
                wr = wr + cm[r, b] * bb_ref[b]
            xt_ref[r] = jnp.dot(h1, wr, preferred_element_type=jnp.float32)
    return pl.pallas_call(
        body,
        grid=(_N // _BN,),
        in_specs=[
            pl.BlockSpec((_NC, _BN, _C), lambda i: (0, i, 0)),
            pl.BlockSpec((_BN, _C), lambda i: (i, 0)),
            pl.BlockSpec((1, _C), lambda i: (0, 0)),
            pl.BlockSpec((_R, _NB), lambda i: (0, 0)),
            pl.BlockSpec((_NB, _C, _C), lambda i: (0, 0, 0)),
        ],
        out_specs=[
            pl.BlockSpec((_BN, _C), lambda i: (i, 0)),
            pl.BlockSpec((_R, _BN, _C), lambda i: (0, i, 0)),
        ],
        out_shape=[
            jax.ShapeDtypeStruct((_N, _C), jnp.float32),
            jax.ShapeDtypeStruct((_R, _N, _C), jnp.float32),
        ],
    )(sc1, root1, bias1, comp2, bases2)


def _tc_layer3(sc2, h1, root2, bias2, comp3, bases3):
    def body(sc_ref, h1_ref, rt_ref, bs_ref, cm_ref, bb_ref, h2_ref, xt_ref):
        h1 = h1_ref[...]
        h2 = jnp.maximum(
            sc_ref[0] + sc_ref[1]
            + jnp.dot(h1, rt_ref[...], preferred_element_type=jnp.float32)
            + bs_ref[...], 0.0)
        h2_ref[...] = h2
        f2 = jnp.concatenate([h1, h2], axis=-1)
        cm = cm_ref[...]
        for r in range(_R):
            wr = cm[r, 0] * bb_ref[0]
            for b in range(1, _NB):
                wr = wr + cm[r, b] * bb_ref[b]
            xt_ref[r] = jnp.dot(f2, wr, preferred_element_type=jnp.float32)
    return pl.pallas_call(
        body,
        grid=(_N // _BN,),
        in_specs=[
            pl.BlockSpec((_NC, _BN, _C), lambda i: (0, i, 0)),
            pl.BlockSpec((_BN, _C), lambda i: (i, 0)),
            pl.BlockSpec((_C, _C), lambda i: (0, 0)),
            pl.BlockSpec((1, _C), lambda i: (0, 0)),
            pl.BlockSpec((_R, _NB), lambda i: (0, 0)),
            pl.BlockSpec((_NB, 2 * _C, _C), lambda i: (0, 0, 0)),
        ],
        out_specs=[
            pl.BlockSpec((_BN, _C), lambda i: (i, 0)),
            pl.BlockSpec((_R, _BN, _C), lambda i: (0, i, 0)),
        ],
        out_shape=[
            jax.ShapeDtypeStruct((_N, _C), jnp.float32),
            jax.ShapeDtypeStruct((_R, _N, _C), jnp.float32),
        ],
    )(sc2, h1, root2, bias2, comp3, bases3)


def _tc_final(sc3, h1, h2, root3, bias3):
    def body(sc_ref, h1_ref, h2_ref, rt_ref, bs_ref, o_ref):
        h1 = h1_ref[...]
        h2 = h2_ref[...]
        rt = rt_ref[...]
        h3 = jnp.maximum(
            sc_ref[0] + sc_ref[1]
            + jnp.dot(h1, rt[:_C], preferred_element_type=jnp.float32)
            + jnp.dot(h2, rt[_C:], preferred_element_type=jnp.float32)
            + bs_ref[...], 0.0)
        f3 = jnp.concatenate([h1, h2, h3], axis=-1)
        m = jnp.max(f3, axis=-1, keepdims=True)
        lse = jnp.log(jnp.sum(jnp.exp(f3 - m), axis=-1, keepdims=True)) + m
        o_ref[...] = f3 - lse
    return pl.pallas_call(
        body,
        grid=(_N // _BN,),
        in_specs=[
            pl.BlockSpec((_NC, _BN, _C), lambda i: (0, i, 0)),
            pl.BlockSpec((_BN, _C), lambda i: (i, 0)),
            pl.BlockSpec((_BN, _C), lambda i: (i, 0)),
            pl.BlockSpec((2 * _C, _C), lambda i: (0, 0)),
            pl.BlockSpec((1, _C), lambda i: (0, 0)),
        ],
        out_specs=pl.BlockSpec((_BN, 3 * _C), lambda i: (i, 0)),
        out_shape=jax.ShapeDtypeStruct((_N, 3 * _C), jnp.float32),
    )(sc3, h1, h2, root3, bias3)


def kernel(edge_index, edge_type, comp1, bases1, root1, bias1,
           comp2, bases2, root2, bias2, comp3, bases3, root3, bias3):
    ei = edge_index.astype(jnp.int32)
    et = edge_type.astype(jnp.int32)
    src = ei[0]
    dst = ei[1]

    g13, w = _sc_prologue(src, dst, et)
    w1 = _tc_table1(comp1, bases1).reshape(_R * _N, _C)
    sc1 = _sc_layer(w1, g13, dst, w)
    h1, xt2 = _tc_layer2(sc1, root1, bias1.reshape(1, _C), comp2, bases2)
    sc2 = _sc_layer(xt2.reshape(_R * _N, _C), g13, dst, w)
    h2, xt3 = _tc_layer3(sc2, h1, root2, bias2.reshape(1, _C), comp3, bases3)
    sc3 = _sc_layer(xt3.reshape(_R * _N, _C), g13, dst, w)
    return _tc_final(sc3, h1, h2, root3, bias3.reshape(1, _C))


# speedup vs baseline: 32.9694x; 1.0018x over previous
import functools

import jax
import jax.numpy as jnp
from jax import lax
from jax.experimental import pallas as pl
from jax.experimental.pallas import tpu as pltpu
from jax.experimental.pallas import tpu_sc as plsc

_N = 10000
_E = 320000
_R = 8
_NB = 4
_C = 128

_NC = 2
_NS = 16
_NW = _NC * _NS
_EPW = _E // _NW
_B = 80
_NBATCH = _EPW // _B
_EPT = _E // _NS
_NCB = _EPT // _B
_KPAD = 81920
_KSL = _KPAD // _NS

_BN = 1000


def _mesh():
    return plsc.VectorSubcoreMesh(core_axis_name="c", subcore_axis_name="s")


def _sc_prologue(src, dst, et):
    @functools.partial(
        pl.kernel,
        out_type=[jax.ShapeDtypeStruct((_E,), jnp.int32),
                  jax.ShapeDtypeStruct((_E,), jnp.float32)],
        mesh=_mesh(),
        scratch_types=[
            pltpu.VMEM((_EPT,), jnp.int32),
            pltpu.VMEM((_EPT,), jnp.int32),
            pltpu.VMEM((_EPW,), jnp.int32),
            pltpu.VMEM((_EPW,), jnp.int32),
            pltpu.VMEM((_EPW,), jnp.int32),
            pltpu.VMEM((_EPW,), jnp.int32),
            pltpu.VMEM((_EPW + 16,), jnp.float32),
            pltpu.VMEM((_B,), jnp.int32),
            pltpu.VMEM((_B,), jnp.int32),
            pltpu.VMEM((_B,), jnp.float32),
            pltpu.VMEM((_KSL,), jnp.float32),
            pltpu.VMEM_SHARED((_KPAD,), jnp.float32),
            pltpu.SemaphoreType.DMA,
            pltpu.SemaphoreType.DMA,
        ],
    )
    def kfn(src_h, dst_h, et_h, g13_o, w_o,
            dstc, typc, srcp, dstp, typp, g13b, wb,
            key0, key1, ones, sbuf, cnt_sh, k0, k1):
        c = lax.axis_index("c")
        s = lax.axis_index("s")
        wid = s * _NC + c

        def zfill(i, carry):
            sbuf[pl.ds(i * 16, 16)] = jnp.zeros((16,), jnp.float32)
            return carry
        lax.fori_loop(0, _KSL // 16, zfill, 0)
        for j in range(_B // 16):
            ones[pl.ds(j * 16, 16)] = jnp.ones((16,), jnp.float32)
        pltpu.sync_copy(sbuf, cnt_sh.at[pl.ds(s * _KSL, _KSL)])
        pltpu.sync_copy(dst_h.at[pl.ds(s * _EPT, _EPT)], dstc)
        pltpu.sync_copy(et_h.at[pl.ds(s * _EPT, _EPT)], typc)
        plsc.subcore_barrier()

        def ckeys(a, key):
            for q in range(_B // 16):
                d = dstc[pl.ds(a * _B + q * 16, 16)]
                t = typc[pl.ds(a * _B + q * 16, 16)]
                key[pl.ds(q * 16, 16)] = d * _R + t

        def cstart(key, sem):
            pltpu.async_copy(ones, cnt_sh.at[key], sem, add=True)

        def cwait(key, sem):
            pltpu.make_async_copy(ones, cnt_sh.at[key], sem).wait()

        ckeys(0, key0)
        cstart(key0, k0)
        ckeys(1, key1)
        cstart(key1, k1)

        def cbody(k, carry):
            a = k * 2
            cwait(key0, k0)
            ckeys(a, key0)
            cstart(key0, k0)
            cwait(key1, k1)
            ckeys(a + 1, key1)
            cstart(key1, k1)
            return carry
        lax.fori_loop(1, _NCB // 2, cbody, 0)
        cwait(key0, k0)
        cwait(key1, k1)
        plsc.subcore_barrier()

        pltpu.sync_copy(cnt_sh.at[pl.ds(s * _KSL, _KSL)], sbuf)

        def rbody(i, carry):
            x = sbuf[pl.ds(i * 16, 16)]
            sbuf[pl.ds(i * 16, 16)] = 1.0 / jnp.maximum(x, 1.0)
            return carry
        lax.fori_loop(0, _KSL // 16, rbody, 0)
        pltpu.sync_copy(sbuf, cnt_sh.at[pl.ds(s * _KSL, _KSL)])
        plsc.subcore_barrier()

        pltpu.sync_copy(src_h.at[pl.ds(wid * _EPW, _EPW)], srcp)
        pltpu.sync_copy(dst_h.at[pl.ds(wid * _EPW, _EPW)], dstp)
        pltpu.sync_copy(et_h.at[pl.ds(wid * _EPW, _EPW)], typp)

        def pkeys(a, key):
            for q in range(_B // 16):
                sj = srcp[pl.ds(a * _B + q * 16, 16)]
                dj = dstp[pl.ds(a * _B + q * 16, 16)]
                tj = typp[pl.ds(a * _B + q * 16, 16)]
                key[pl.ds(q * 16, 16)] = dj * _R + tj
                g13b[pl.ds(a * _B + q * 16, 16)] = tj * _N + sj

        def gstart(a, key, sem):
            pltpu.async_copy(cnt_sh.at[key], wb.at[pl.ds(a * _B, _B)], sem)

        def gwait(key, sem):
            pltpu.make_async_copy(cnt_sh.at[key], wb.at[pl.ds(0, _B)],
                                  sem).wait()

        pkeys(0, key0)
        gstart(0, key0, k0)
        pkeys(1, key1)
        gstart(1, key1, k1)

        def pbody(k, carry):
            a = k * 2
            gwait(key0, k0)
            pkeys(a, key0)
            gstart(a, key0, k0)
            gwait(key1, k1)
            pkeys(a + 1, key1)
            gstart(a + 1, key1, k1)
            return carry
        lax.fori_loop(1, 62, pbody, 0)
        gwait(key0, k0)
        pkeys(124, key0)
        gstart(124, key0, k0)
        gwait(key1, k1)
        gwait(key0, k0)
        pltpu.sync_copy(g13b, g13_o.at[pl.ds(wid * _EPW, _EPW)])
        pltpu.sync_copy(wb.at[pl.ds(0, _EPW)], w_o.at[pl.ds(wid * _EPW, _EPW)])

    return kfn(src, dst, et)


_LB = 40
_LNB = _EPW // _LB


def _sc_layer(table, g13, dst, w):
    @functools.partial(
        pl.kernel,
        out_type=jax.ShapeDtypeStruct((_NC, _N, _C), jnp.float32),
        mesh=_mesh(),
        scratch_types=[
            pltpu.VMEM((_EPW,), jnp.int32),
            pltpu.VMEM((_EPW + 16,), jnp.float32),
            pltpu.VMEM((_EPW,), jnp.int32),
            pltpu.VMEM((_LB,), jnp.int32),
            pltpu.VMEM((_LB,), jnp.int32),
            pltpu.VMEM((_LB,), jnp.int32),
            pltpu.VMEM((_LB, _C), jnp.float32),
            pltpu.VMEM((_LB, _C), jnp.float32),
            pltpu.VMEM((_LB, _C), jnp.float32),
            pltpu.VMEM_SHARED((_N, _C), jnp.float32),
            pltpu.SemaphoreType.DMA,
            pltpu.SemaphoreType.DMA,
            pltpu.SemaphoreType.DMA,
            pltpu.SemaphoreType.DMA,
            pltpu.SemaphoreType.DMA,
            pltpu.SemaphoreType.DMA,
        ],
    )
    def kfn(table_h, g13_h, dst_h, w_h, out_h,
            gidx_all, w_all, dst_all, dstb0, dstb1, dstb2,
            rows0, rows1, rows2, acc, g0, g1, g2, s0, s1, s2):
        c = lax.axis_index("c")
        s = lax.axis_index("s")
        wid = s * _NC + c
        row_base = s * 632
        ebase = wid * _EPW
        rows = (rows0, rows1, rows2)
        dstb = (dstb0, dstb1, dstb2)
        gsem = (g0, g1, g2)
        ssem = (s0, s1, s2)

        pltpu.sync_copy(g13_h.at[pl.ds(ebase, _EPW)], gidx_all)
        pltpu.sync_copy(w_h.at[pl.ds(ebase, _EPW)], w_all.at[pl.ds(0, _EPW)])
        pltpu.sync_copy(dst_h.at[pl.ds(ebase, _EPW)], dst_all)

        def zfill(i, carry):
            for j in range(_C // 16):
                rows0[i, pl.ds(j * 16, 16)] = jnp.zeros((16,), jnp.float32)
            return carry
        lax.fori_loop(0, _LB, zfill, 0)
        nfull = lax.select(s == _NS - 1, 13, 15)

        def zcopy(k, carry):
            pltpu.sync_copy(rows0, acc.at[pl.ds(row_base + k * _LB, _LB), :])
            return carry
        lax.fori_loop(0, nfull, zcopy, 0)

        @pl.when(s != _NS - 1)
        def _ztail():
            pltpu.sync_copy(rows0.at[pl.ds(0, 32), :],
                            acc.at[pl.ds(row_base + 600, 32), :])
        plsc.subcore_barrier()

        def gstart(ib, u):
            pltpu.async_copy(
                table_h.at[gidx_all.at[pl.ds(ib * _LB, _LB)]], rows[u],
                gsem[u])

        def gwait(u):
            pltpu.make_async_copy(table_h.at[gidx_all.at[pl.ds(0, _LB)]],
                                  rows[u], gsem[u]).wait()

        def sstart(ib, u):
            for o in (0, 16, 24):
                dstb[u][pl.ds(o, 16)] = dst_all[pl.ds(ib * _LB + o, 16)]
            pltpu.async_copy(rows[u], acc.at[dstb[u]], ssem[u], add=True)

        def swait(u):
            pltpu.make_async_copy(rows[u], acc.at[dstb0], ssem[u]).wait()

        def scale(ib, u):
            for q in range(_LB // 16):
                wv = w_all[pl.ds(ib * _LB + q * 16, 16)]
                for i in range(16):
                    e = q * 16 + i
                    ws = wv[i]
                    for j in range(_C // 16):
                        rows[u][e, pl.ds(j * 16, 16)] = (
                            rows[u][e, pl.ds(j * 16, 16)] * ws)
            wv = w_all[pl.ds(ib * _LB + 32, 16)]
            for i in range(8):
                e = 32 + i
                ws = wv[i]
                for j in range(_C // 16):
                    rows[u][e, pl.ds(j * 16, 16)] = (
                        rows[u][e, pl.ds(j * 16, 16)] * ws)

        gstart(0, 0)
        gstart(1, 1)

        def body(k, carry):
            for u in range(3):
                ib = k * 3 + u
                z = (u + 2) % 3
                gwait(u)
                scale(ib, u)
                sstart(ib, u)

                @pl.when(ib >= 1)
                def _():
                    swait(z)

                @pl.when(ib + 2 < _LNB)
                def _():
                    gstart(ib + 2, z)
            return carry
        lax.fori_loop(0, (_LNB - 1) // 3, body, 0)
        gwait(0)
        scale(_LNB - 1, 0)
        sstart(_LNB - 1, 0)
        swait(2)
        swait(0)
        plsc.subcore_barrier()

        @pl.when(s != _NS - 1)
        def _drain_full():
            pltpu.sync_copy(acc.at[pl.ds(row_base, 632), :],
                            out_h.at[c, pl.ds(row_base, 632), :])

        @pl.when(s == _NS - 1)
        def _drain_tail():
            pltpu.sync_copy(acc.at[pl.ds(15 * 632, 520), :],
                            out_h.at[c, pl.ds(15 * 632, 520), :])

    return kfn(table, g13, dst, w)


def _tc_table1(comp1, bases1):
    def body(cm_ref, bb_ref, o_ref):
        cm = cm_ref[...]
        for r in range(_R):
            acc = cm[r, 0] * bb_ref[0]
            for b in range(1, _NB):
                acc = acc + cm[r, b] * bb_ref[b]
            o_ref[r] = acc
    return pl.pallas_call(
        body,
        grid=(_N // _BN,),
        in_specs=[
            pl.BlockSpec((_R, _NB), lambda i: (0, 0)),
            pl.BlockSpec((_NB, _BN, _C), lambda i: (0, i, 0)),
        ],
        out_specs=pl.BlockSpec((_R, _BN, _C), lambda i: (0, i, 0)),
        out_shape=jax.ShapeDtypeStruct((_R, _N, _C), jnp.float32),
    )(comp1, bases1)


def _tc_layer2(sc1, root1, bias1, comp2, bases2):
    def body(sc_ref, rt_ref, bs_ref, cm_ref, bb_ref, h1_ref, xt_ref):
        h1 = jnp.maximum(sc_ref[0] + sc_ref[1] + rt_ref[...] + bs_ref[...], 0.0)
        h1_ref[...] = h1
        cm = cm_ref[...]
        for r in range(_R):
            wr = cm[r, 0] * bb_ref[0]
            for b in range(1, _NB):
                wr = wr + cm[r, b] * bb_ref[b]
            xt_ref[r] = jnp.dot(h1, wr, preferred_element_type=jnp.float32)
    return pl.pallas_call(
        body,
        grid=(_N // _BN,),
        in_specs=[
            pl.BlockSpec((_NC, _BN, _C), lambda i: (0, i, 0)),
            pl.BlockSpec((_BN, _C), lambda i: (i, 0)),
            pl.BlockSpec((1, _C), lambda i: (0, 0)),
            pl.BlockSpec((_R, _NB), lambda i: (0, 0)),
            pl.BlockSpec((_NB, _C, _C), lambda i: (0, 0, 0)),
        ],
        out_specs=[
            pl.BlockSpec((_BN, _C), lambda i: (i, 0)),
            pl.BlockSpec((_R, _BN, _C), lambda i: (0, i, 0)),
        ],
        out_shape=[
            jax.ShapeDtypeStruct((_N, _C), jnp.float32),
            jax.ShapeDtypeStruct((_R, _N, _C), jnp.float32),
        ],
    )(sc1, root1, bias1, comp2, bases2)


def _tc_layer3(sc2, h1, root2, bias2, comp3, bases3):
    def body(sc_ref, h1_ref, rt_ref, bs_ref, cm_ref, bb_ref, h2_ref, xt_ref):
        h1 = h1_ref[...]
        h2 = jnp.maximum(
            sc_ref[0] + sc_ref[1]
            + jnp.dot(h1, rt_ref[...], preferred_element_type=jnp.float32)
            + bs_ref[...], 0.0)
        h2_ref[...] = h2
        f2 = jnp.concatenate([h1, h2], axis=-1)
        cm = cm_ref[...]
        for r in range(_R):
            wr = cm[r, 0] * bb_ref[0]
            for b in range(1, _NB):
                wr = wr + cm[r, b] * bb_ref[b]
            xt_ref[r] = jnp.dot(f2, wr, preferred_element_type=jnp.float32)
    return pl.pallas_call(
        body,
        grid=(_N // _BN,),
        in_specs=[
            pl.BlockSpec((_NC, _BN, _C), lambda i: (0, i, 0)),
            pl.BlockSpec((_BN, _C), lambda i: (i, 0)),
            pl.BlockSpec((_C, _C), lambda i: (0, 0)),
            pl.BlockSpec((1, _C), lambda i: (0, 0)),
            pl.BlockSpec((_R, _NB), lambda i: (0, 0)),
            pl.BlockSpec((_NB, 2 * _C, _C), lambda i: (0, 0, 0)),
        ],
        out_specs=[
            pl.BlockSpec((_BN, _C), lambda i: (i, 0)),
            pl.BlockSpec((_R, _BN, _C), lambda i: (0, i, 0)),
        ],
        out_shape=[
            jax.ShapeDtypeStruct((_N, _C), jnp.float32),
            jax.ShapeDtypeStruct((_R, _N, _C), jnp.float32),
        ],
    )(sc2, h1, root2, bias2, comp3, bases3)


def _tc_final(sc3, h1, h2, root3, bias3):
    def body(sc_ref, h1_ref, h2_ref, rt_ref, bs_ref, o_ref):
        h1 = h1_ref[...]
        h2 = h2_ref[...]
        rt = rt_ref[...]
        h3 = jnp.maximum(
            sc_ref[0] + sc_ref[1]
            + jnp.dot(h1, rt[:_C], preferred_element_type=jnp.float32)
            + jnp.dot(h2, rt[_C:], preferred_element_type=jnp.float32)
            + bs_ref[...], 0.0)
        f3 = jnp.concatenate([h1, h2, h3], axis=-1)
        m = jnp.max(f3, axis=-1, keepdims=True)
        lse = jnp.log(jnp.sum(jnp.exp(f3 - m), axis=-1, keepdims=True)) + m
        o_ref[...] = f3 - lse
    return pl.pallas_call(
        body,
        grid=(_N // _BN,),
        in_specs=[
            pl.BlockSpec((_NC, _BN, _C), lambda i: (0, i, 0)),
            pl.BlockSpec((_BN, _C), lambda i: (i, 0)),
            pl.BlockSpec((_BN, _C), lambda i: (i, 0)),
            pl.BlockSpec((2 * _C, _C), lambda i: (0, 0)),
            pl.BlockSpec((1, _C), lambda i: (0, 0)),
        ],
        out_specs=pl.BlockSpec((_BN, 3 * _C), lambda i: (i, 0)),
        out_shape=jax.ShapeDtypeStruct((_N, 3 * _C), jnp.float32),
    )(sc3, h1, h2, root3, bias3)


def kernel(edge_index, edge_type, comp1, bases1, root1, bias1,
           comp2, bases2, root2, bias2, comp3, bases3, root3, bias3):
    ei = edge_index.astype(jnp.int32)
    et = edge_type.astype(jnp.int32)
    src = ei[0]
    dst = ei[1]

    g13, w = _sc_prologue(src, dst, et)
    w1 = _tc_table1(comp1, bases1).reshape(_R * _N, _C)
    sc1 = _sc_layer(w1, g13, dst, w)
    h1, xt2 = _tc_layer2(sc1, root1, bias1.reshape(1, _C), comp2, bases2)
    sc2 = _sc_layer(xt2.reshape(_R * _N, _C), g13, dst, w)
    h2, xt3 = _tc_layer3(sc2, h1, root2, bias2.reshape(1, _C), comp3, bases3)
    sc3 = _sc_layer(xt3.reshape(_R * _N, _C), g13, dst, w)
    return _tc_final(sc3, h1, h2, root3, bias3.reshape(1, _C))
